# Initial kernel scaffold; baseline (speedup 1.0000x reference)
import jax
import jax.numpy as jnp
from jax.experimental import pallas as pl


def kernel(X, A, W1, b1, W2, b2):
    raise NotImplementedError("write your pallas kernel here")


# speedup vs baseline: 1451.1914x; 1451.1914x over previous
import jax
import jax.numpy as jnp
from jax.experimental import pallas as pl


def _l2norm(x):
    n = jnp.sqrt(jnp.sum(x * x, axis=-1, keepdims=True))
    return x / jnp.maximum(n, 1e-12)


def _net_kernel(x_ref, a_ref, w1_ref, b1_ref, w2_ref, b2_ref, o_ref):
    n = a_ref.shape[0]
    a = a_ref[...]
    row = jax.lax.broadcasted_iota(jnp.int32, (n, n), 0)
    col = jax.lax.broadcasted_iota(jnp.int32, (n, n), 1)
    af = jnp.where((row == col) | a, 1.0, 0.0).astype(jnp.float32)

    tdot = lambda m, y: jax.lax.dot_general(
        m, y, (((0,), (0,)), ((), ())), preferred_element_type=jnp.float32
    )

    y1 = jnp.dot(x_ref[...], w1_ref[...], preferred_element_type=jnp.float32)
    h = _l2norm(tdot(af, y1) + b1_ref[...])
    h = jnp.maximum(h, 0.0)

    y2 = jnp.dot(h, w2_ref[...], preferred_element_type=jnp.float32)
    o = _l2norm(tdot(af, y2) + b2_ref[...])
    o_ref[...] = _l2norm(o)


def kernel(X, A, W1, b1, W2, b2):
    n = X.shape[0]
    h = W1.shape[1]
    return pl.pallas_call(
        _net_kernel,
        out_shape=jax.ShapeDtypeStruct((n, h), jnp.float32),
    )(X, A, W1, b1.reshape(1, h), W2, b2.reshape(1, h))
